# Initial kernel scaffold; baseline (speedup 1.0000x reference)
import jax
import jax.numpy as jnp
from jax.experimental import pallas as pl


def kernel(x, preds, W1, b1, W2, b2):
    raise NotImplementedError("write your pallas kernel here")


# speedup vs baseline: 6.4976x; 6.4976x over previous
import jax
import jax.numpy as jnp
from jax.experimental import pallas as pl
from jax.experimental.pallas import tpu as pltpu

_NC = 19
_NCP = 24
_T = 4096


def _body(x_ref, p_ref, w1_ref, b1_ref, w2_ref, b2_ref, out_ref,
          nw_ref, at_ref, m_ref, d_ref):
    j = pl.program_id(1)
    nblk = pl.num_programs(1)
    c = x_ref.shape[1]

    @pl.when(j == 0)
    def _init():
        nw_ref[...] = jnp.zeros_like(nw_ref)
        at_ref[...] = jnp.zeros_like(at_ref)
        d_ref[...] = jnp.zeros_like(d_ref)
        m_ref[...] = jnp.full_like(m_ref, -jnp.inf)

    p = p_ref[0]
    x = x_ref[0]
    m = jnp.max(p, axis=0, keepdims=True)
    idx = jax.lax.broadcasted_iota(jnp.int32, (_NC, _T), 0)
    cand = jnp.where(p == m, idx, _NC)
    k = jnp.min(cand, axis=0, keepdims=True)
    rows = jax.lax.broadcasted_iota(jnp.int32, (_NCP, _T), 0)
    sel = rows == k
    maskf = sel.astype(jnp.float32)

    mb = jnp.where(sel, jnp.broadcast_to(m, (_NCP, _T)), -jnp.inf)
    pm = jnp.max(mb, axis=1, keepdims=True)
    m_old = m_ref[...]
    m_new = jnp.maximum(m_old, pm)
    resc = jnp.where(m_old > -jnp.inf, jnp.exp(m_old - m_new), 0.0)

    m_new_col = m_new[:, 0:1]
    m_safe = jnp.where(m_new_col > -jnp.inf, m_new_col, 0.0)
    m_pp = jax.lax.dot_general(m_safe, maskf, (((0,), (0,)), ((), ())),
                               precision=jax.lax.Precision.HIGHEST,
                               preferred_element_type=jnp.float32)
    w = jnp.exp(m - m_pp)
    wm = maskf * w

    r = jax.lax.dot_general(wm, x, (((1,), (1,)), ((), ())),
                            precision=jax.lax.Precision.HIGHEST,
                            preferred_element_type=jnp.float32)
    se = jnp.sum(wm, axis=1, keepdims=True)

    x1 = jax.lax.dot_general(
        w1_ref[...].astype(jnp.bfloat16), x.astype(jnp.bfloat16),
        (((1,), (0,)), ((), ())), preferred_element_type=jnp.float32)
    x1 = x1 + b1_ref[:, 0:1]
    ra = jax.lax.dot_general(
        maskf.astype(jnp.bfloat16), x1.astype(jnp.bfloat16),
        (((1,), (1,)), ((), ())),
        preferred_element_type=jnp.float32)

    nw_ref[...] = nw_ref[...] * resc + r
    at_ref[...] = at_ref[...] + ra
    d_ref[...] = d_ref[...] * resc + se
    m_ref[...] = m_new

    @pl.when(j == nblk - 1)
    def _fin():
        d = d_ref[...]
        d_safe = jnp.where(d > 0, d, 1.0)
        cls_feat = nw_ref[...] / d_safe
        cf2 = jax.lax.dot_general(
            cls_feat.astype(jnp.bfloat16), w2_ref[...].astype(jnp.bfloat16),
            (((1,), (1,)), ((), ())),
            preferred_element_type=jnp.float32) + b2_ref[...]
        cf2 = cf2.astype(jnp.bfloat16).astype(jnp.float32)
        gc = jax.lax.dot_general(at_ref[...], cf2, (((0,), (0,)), ((), ())),
                                 precision=jax.lax.Precision.HIGHEST,
                                 preferred_element_type=jnp.float32)
        gc = gc * (c ** -0.5)
        gmax = jnp.max(gc, axis=1, keepdims=True)
        e = jnp.exp(gc - gmax)
        out_ref[0] = e / jnp.sum(e, axis=1, keepdims=True)


def kernel(x, preds, W1, b1, W2, b2):
    b, c, h, w = x.shape
    hw = h * w
    nc = preds.shape[1]
    x2 = x.reshape(b, c, hw)
    p2 = preds.reshape(b, nc, hw)
    nblk = hw // _T
    b1t = jnp.broadcast_to(b1[:, None], (c, 128))
    out = pl.pallas_call(
        _body,
        grid=(b, nblk),
        in_specs=[
            pl.BlockSpec((1, c, _T), lambda i, j: (i, 0, j)),
            pl.BlockSpec((1, nc, _T), lambda i, j: (i, 0, j)),
            pl.BlockSpec((c, c), lambda i, j: (0, 0)),
            pl.BlockSpec((c, 128), lambda i, j: (0, 0)),
            pl.BlockSpec((c, c), lambda i, j: (0, 0)),
            pl.BlockSpec((1, c), lambda i, j: (0, 0)),
        ],
        out_specs=pl.BlockSpec((1, c, c), lambda i, j: (i, 0, 0)),
        out_shape=jax.ShapeDtypeStruct((b, c, c), jnp.float32),
        scratch_shapes=[pltpu.VMEM((_NCP, 128), jnp.float32)] * 4,
        compiler_params=pltpu.CompilerParams(
            dimension_semantics=("arbitrary", "arbitrary")),
    )(x2, p2, W1, b1t, W2, b2.reshape(1, c))
    return out


# speedup vs baseline: 8.8920x; 1.3685x over previous
import jax
import jax.numpy as jnp
from jax.experimental import pallas as pl
from jax.experimental.pallas import tpu as pltpu

_NC = 19
_NCP = 24
_T = 8192


def _body(x_ref, p_ref, w1_ref, b1_ref, w2_ref, b2_ref, out_ref,
          nw_ref, at_ref, m_ref, d_ref):
    j = pl.program_id(1)
    nblk = pl.num_programs(1)
    c = x_ref.shape[1]

    @pl.when(j == 0)
    def _init():
        nw_ref[...] = jnp.zeros_like(nw_ref)
        at_ref[...] = jnp.zeros_like(at_ref)
        d_ref[...] = jnp.zeros_like(d_ref)
        m_ref[...] = jnp.full_like(m_ref, -jnp.inf)

    p = p_ref[0]
    x = x_ref[0]
    m = jnp.max(p, axis=0, keepdims=True)
    idx = jax.lax.broadcasted_iota(jnp.int32, (_NC, _T), 0)
    cand = jnp.where(p == m, idx, _NC)
    k = jnp.min(cand, axis=0, keepdims=True)
    rows = jax.lax.broadcasted_iota(jnp.int32, (_NCP, _T), 0)
    sel = rows == k
    maskf = sel.astype(jnp.float32)

    mb = jnp.where(sel, jnp.broadcast_to(m, (_NCP, _T)), -jnp.inf)
    pm = jnp.max(mb, axis=1, keepdims=True)
    m_old = m_ref[...]
    m_new = jnp.maximum(m_old, pm)
    resc = jnp.where(m_old > -jnp.inf, jnp.exp(m_old - m_new), 0.0)

    m_new_col = m_new[:, 0:1]
    mv = jnp.where(sel, jnp.broadcast_to(m_new_col, (_NCP, _T)), -jnp.inf)
    m_pp = jnp.max(mv, axis=0, keepdims=True)
    w = jnp.exp(m - m_pp)
    wm = maskf * w

    x_bf = x.astype(jnp.bfloat16)
    r = jax.lax.dot_general(wm.astype(jnp.bfloat16), x_bf,
                            (((1,), (1,)), ((), ())),
                            preferred_element_type=jnp.float32)
    se = jnp.sum(wm, axis=1, keepdims=True)

    x1 = jax.lax.dot_general(
        w1_ref[...].astype(jnp.bfloat16), x_bf,
        (((1,), (0,)), ((), ())), preferred_element_type=jnp.float32)
    x1 = x1 + b1_ref[:, 0:1]
    ra = jax.lax.dot_general(
        maskf.astype(jnp.bfloat16), x1.astype(jnp.bfloat16),
        (((1,), (1,)), ((), ())),
        preferred_element_type=jnp.float32)

    nw_ref[...] = nw_ref[...] * resc + r
    at_ref[...] = at_ref[...] + ra
    d_ref[...] = d_ref[...] * resc + se
    m_ref[...] = m_new

    @pl.when(j == nblk - 1)
    def _fin():
        d = d_ref[...]
        d_safe = jnp.where(d > 0, d, 1.0)
        cls_feat = nw_ref[...] / d_safe
        cf2 = jax.lax.dot_general(
            cls_feat.astype(jnp.bfloat16), w2_ref[...].astype(jnp.bfloat16),
            (((1,), (1,)), ((), ())),
            preferred_element_type=jnp.float32) + b2_ref[...]
        cf2 = cf2.astype(jnp.bfloat16).astype(jnp.float32)
        gc = jax.lax.dot_general(at_ref[...], cf2, (((0,), (0,)), ((), ())),
                                 precision=jax.lax.Precision.HIGHEST,
                                 preferred_element_type=jnp.float32)
        gc = gc * (c ** -0.5)
        gmax = jnp.max(gc, axis=1, keepdims=True)
        e = jnp.exp(gc - gmax)
        out_ref[0] = e / jnp.sum(e, axis=1, keepdims=True)


def kernel(x, preds, W1, b1, W2, b2):
    b, c, h, w = x.shape
    hw = h * w
    nc = preds.shape[1]
    x2 = x.reshape(b, c, hw)
    p2 = preds.reshape(b, nc, hw)
    nblk = hw // _T
    b1t = jnp.broadcast_to(b1[:, None], (c, 128))
    out = pl.pallas_call(
        _body,
        grid=(b, nblk),
        in_specs=[
            pl.BlockSpec((1, c, _T), lambda i, j: (i, 0, j)),
            pl.BlockSpec((1, nc, _T), lambda i, j: (i, 0, j)),
            pl.BlockSpec((c, c), lambda i, j: (0, 0)),
            pl.BlockSpec((c, 128), lambda i, j: (0, 0)),
            pl.BlockSpec((c, c), lambda i, j: (0, 0)),
            pl.BlockSpec((1, c), lambda i, j: (0, 0)),
        ],
        out_specs=pl.BlockSpec((1, c, c), lambda i, j: (i, 0, 0)),
        out_shape=jax.ShapeDtypeStruct((b, c, c), jnp.float32),
        scratch_shapes=[pltpu.VMEM((_NCP, 128), jnp.float32)] * 4,
        compiler_params=pltpu.CompilerParams(
            dimension_semantics=("arbitrary", "arbitrary")),
    )(x2, p2, W1, b1t, W2, b2.reshape(1, c))
    return out
